# baseline (speedup 1.0000x reference)
import functools

import numpy as np

import jax
import jax.numpy as jnp
from jax import lax
from jax.experimental import pallas as pl
from jax.experimental.pallas import tpu as pltpu
from jax.experimental.pallas import tpu_sc as plsc

N = 10000
E = 320000
R = 8
D = 128

NC = 2
NS = 16
NW = NC * NS
EPT = E // NW
KTAB = 81920
BC = 2000
BA = 80
PN = 10240

_mesh = plsc.VectorSubcoreMesh(core_axis_name="c", subcore_axis_name="s")


def _zero_flat(ref, nwords):
    z = jnp.zeros((16,), jnp.float32)

    def body(i, carry):
        ref[pl.ds(i * 16, 16)] = z
        return carry

    lax.fori_loop(0, nwords // 16, body, 0)


@functools.partial(
    pl.kernel,
    out_type=jax.ShapeDtypeStruct((NW, KTAB), jnp.float32),
    mesh=_mesh,
    compiler_params=pltpu.CompilerParams(needs_layout_passes=False),
    scratch_types=[
        pltpu.VMEM((KTAB,), jnp.float32),
        pltpu.VMEM((BC,), jnp.int32),
        pltpu.VMEM((BC,), jnp.int32),
    ],
)
def _sc_count(dst_hbm, et_hbm, out_hbm, cnt_l, dbuf, ebuf):
    c = lax.axis_index("c")
    s = lax.axis_index("s")
    wid = s * NC + c
    _zero_flat(cnt_l, KTAB)
    ones = jnp.full((16,), 1.0, jnp.float32)
    base = wid * EPT
    for blk in range(EPT // BC):
        off = base + blk * BC
        pltpu.sync_copy(dst_hbm.at[pl.ds(off, BC)], dbuf)
        pltpu.sync_copy(et_hbm.at[pl.ds(off, BC)], ebuf)

        def cbody(i, carry):
            d = dbuf[pl.ds(i * 16, 16)]
            t = ebuf[pl.ds(i * 16, 16)]
            plsc.addupdate_scatter(cnt_l, [t * N + d], ones)
            return carry

        lax.fori_loop(0, BC // 16, cbody, 0)
    pltpu.sync_copy(cnt_l, out_hbm.at[wid])


_IW = KTAB // NW


@functools.partial(
    pl.kernel,
    out_type=jax.ShapeDtypeStruct((KTAB,), jnp.float32),
    mesh=_mesh,
    compiler_params=pltpu.CompilerParams(needs_layout_passes=False),
    scratch_types=[
        pltpu.VMEM((_IW,), jnp.float32),
        pltpu.VMEM((_IW,), jnp.float32),
    ],
)
def _sc_inv(parts_hbm, inv_hbm, acc, tmp):
    c = lax.axis_index("c")
    s = lax.axis_index("s")
    wid = s * NC + c
    r0 = wid * _IW
    pltpu.sync_copy(parts_hbm.at[0, pl.ds(r0, _IW)], acc)
    for p in range(1, NW):
        pltpu.sync_copy(parts_hbm.at[p, pl.ds(r0, _IW)], tmp)

        def abody(i, carry):
            sl = pl.ds(i * 16, 16)
            acc[sl] = acc[sl] + tmp[sl]
            return carry

        lax.fori_loop(0, _IW // 16, abody, 0)

    def ibody(i, carry):
        sl = pl.ds(i * 16, 16)
        acc[sl] = 1.0 / jnp.maximum(acc[sl], 1.0)
        return carry

    lax.fori_loop(0, _IW // 16, ibody, 0)
    pltpu.sync_copy(acc, inv_hbm.at[pl.ds(r0, _IW)])


@functools.partial(
    pl.kernel,
    out_type=(jax.ShapeDtypeStruct((E,), jnp.float32),
              jax.ShapeDtypeStruct((E,), jnp.int32)),
    mesh=_mesh,
    compiler_params=pltpu.CompilerParams(needs_layout_passes=False),
    scratch_types=[
        pltpu.VMEM((KTAB,), jnp.float32),
        pltpu.VMEM((BC,), jnp.int32),
        pltpu.VMEM((BC,), jnp.int32),
        pltpu.VMEM((BC,), jnp.int32),
        pltpu.VMEM((BC,), jnp.float32),
        pltpu.VMEM((BC,), jnp.int32),
    ],
)
def _sc_scale(inv_hbm, src_hbm, et_hbm, dst_hbm, scale_hbm, rid_hbm,
              inv_l, sbuf, ebuf, dbuf, scblk, ridblk):
    c = lax.axis_index("c")
    s = lax.axis_index("s")
    wid = s * NC + c
    pltpu.sync_copy(inv_hbm, inv_l)
    base = wid * EPT
    for blk in range(EPT // BC):
        off = base + blk * BC
        pltpu.sync_copy(src_hbm.at[pl.ds(off, BC)], sbuf)
        pltpu.sync_copy(et_hbm.at[pl.ds(off, BC)], ebuf)
        pltpu.sync_copy(dst_hbm.at[pl.ds(off, BC)], dbuf)

        def cbody(i, carry):
            sl = pl.ds(i * 16, 16)
            tn = ebuf[sl] * N
            scblk[sl] = plsc.load_gather(inv_l, [tn + dbuf[sl]])
            ridblk[sl] = tn + sbuf[sl]
            return carry

        lax.fori_loop(0, BC // 16, cbody, 0)
        pltpu.sync_copy(scblk, scale_hbm.at[pl.ds(off, BC)])
        pltpu.sync_copy(ridblk, rid_hbm.at[pl.ds(off, BC)])


BB = 128
NBLK = E // BB // NW
SB = 13
SC_CH = NBLK // SB
SE = SB * BB


@functools.partial(
    pl.kernel,
    out_type=jax.ShapeDtypeStruct((NC, PN, 128), jnp.float32),
    mesh=_mesh,
    compiler_params=pltpu.CompilerParams(needs_layout_passes=False),
    scratch_types=[
        pltpu.VMEM((2, BB, 128), jnp.float32),
        pltpu.VMEM((2, SE), jnp.int32),
        pltpu.VMEM((2, SE), jnp.int32),
        pltpu.VMEM((2, SE), jnp.float32),
        pltpu.VMEM((BB,), jnp.int32),
        pltpu.VMEM_SHARED((PN, 128), jnp.float32),
        pltpu.SemaphoreType.DMA,
        pltpu.SemaphoreType.DMA,
        pltpu.SemaphoreType.DMA,
        pltpu.SemaphoreType.DMA,
    ],
)
def _sc_agg(trans_hbm, scale_hbm, rid_hbm, dst_hbm, out_hbm,
            rows, rid_st, dst_st, scale_st, dstb, acc,
            semi0, semi1, semg0, semg1):
    c = lax.axis_index("c")
    s = lax.axis_index("s")
    wid = s * NC + c
    semi = [semi0, semi1]
    semg = [semg0, semg1]

    z = jnp.zeros((16,), jnp.float32)

    def zb(r, carry):
        for k in range(8):
            rows[0, r, pl.ds(k * 16, 16)] = z
        return carry

    lax.fori_loop(0, BB, zb, 0)
    for j in range(5):
        pltpu.sync_copy(rows.at[0], acc.at[pl.ds(s * 640 + j * 128, 128)])
    plsc.subcore_barrier()

    blk0 = wid * NBLK + jnp.minimum(wid, 4)
    e0 = blk0 * BB

    def stage_idx(ci, cb):
        off = e0 + ci * SE
        return (
            pltpu.async_copy(rid_hbm.at[pl.ds(off, SE)], rid_st.at[cb],
                             semi[cb]),
            pltpu.async_copy(dst_hbm.at[pl.ds(off, SE)], dst_st.at[cb],
                             semi[cb]),
            pltpu.async_copy(scale_hbm.at[pl.ds(off, SE)], scale_st.at[cb],
                             semi[cb]),
        )

    def gather(ci, j, p):
        return pltpu.async_copy(
            trans_hbm.at[rid_st.at[ci % 2, pl.ds(j * BB, BB)]],
            rows.at[p], semg[p])

    def scale_rows(cb, j, p):
        def rbody(r2, carry):
            ra = r2 * 2
            rb = ra + 1
            cbv = jnp.full((16,), cb, jnp.int32)
            cola = jnp.full((16,), j * BB, jnp.int32) + ra
            spa = plsc.load_gather(scale_st, [cbv, cola])
            spb = plsc.load_gather(scale_st, [cbv, cola + 1])
            for k in range(8):
                sl = pl.ds(k * 16, 16)
                rows[p, ra, sl] = rows[p, ra, sl] * spa
                rows[p, rb, sl] = rows[p, rb, sl] * spb
            return carry

        lax.fori_loop(0, BB // 2, rbody, 0)

    def scatter(cb, j, p):
        for k in range(8):
            sl = pl.ds(k * 16, 16)
            dstb[sl] = dst_st[cb, pl.ds(j * BB + k * 16, 16)]
        pltpu.sync_copy(rows.at[p], acc.at[dstb], add=True)

    descs = {0: stage_idx(0, 0), 1: stage_idx(1, 1)}
    for ci in range(SC_CH):
        cb = ci % 2
        for d in descs.pop(ci):
            d.wait()
        g = gather(ci, 0, 0)
        for j in range(SB):
            p = j % 2
            g_next = gather(ci, j + 1, 1 - p) if j + 1 < SB else None
            g.wait()
            scale_rows(cb, j, p)
            scatter(cb, j, p)
            g = g_next
        if ci + 2 < SC_CH:
            descs[ci + 2] = stage_idx(ci + 2, cb)

    @pl.when(wid < 4)
    def _extra():
        off = e0 + NBLK * BB
        pltpu.sync_copy(rid_hbm.at[pl.ds(off, BB)], rid_st.at[0, pl.ds(0, BB)])
        pltpu.sync_copy(dst_hbm.at[pl.ds(off, BB)], dstb)
        pltpu.sync_copy(scale_hbm.at[pl.ds(off, BB)],
                        scale_st.at[0, pl.ds(0, BB)])
        pltpu.async_copy(trans_hbm.at[rid_st.at[0, pl.ds(0, BB)]],
                         rows.at[0], semg0).wait()
        scale_rows(0, 0, 0)
        pltpu.sync_copy(rows.at[0], acc.at[dstb], add=True)

    plsc.subcore_barrier()
    for j in range(5):
        r0 = s * 640 + j * 128
        pltpu.sync_copy(acc.at[pl.ds(r0, 128)], out_hbm.at[c, pl.ds(r0, 128)])


_NB = 10
_BM = N // _NB


def _trans_body(x_ref, w_ref, o_ref):
    o_ref[0] = jnp.dot(x_ref[...], w_ref[0],
                       preferred_element_type=jnp.float32)


_dense_trans = pl.pallas_call(
    _trans_body,
    grid=(R, _NB),
    in_specs=[
        pl.BlockSpec((_BM, D), lambda r, i: (i, 0)),
        pl.BlockSpec((1, D, D), lambda r, i: (r, 0, 0)),
    ],
    out_specs=pl.BlockSpec((1, _BM, D), lambda r, i: (r, i, 0)),
    out_shape=jax.ShapeDtypeStruct((R, N, D), jnp.float32),
)


def _comb_body(relu, a0_ref, a1_ref, x_ref, w_ref, b_ref, o_ref):
    acc = (a0_ref[0] + a1_ref[0]
           + jnp.dot(x_ref[...], w_ref[...],
                     preferred_element_type=jnp.float32)
           + b_ref[...])
    if relu:
        acc = jnp.maximum(acc, 0.0)
    o_ref[...] = acc


def _make_combine(relu):
    return pl.pallas_call(
        functools.partial(_comb_body, relu),
        grid=(_NB,),
        in_specs=[
            pl.BlockSpec((1, _BM, D), lambda i: (0, i, 0)),
            pl.BlockSpec((1, _BM, D), lambda i: (1, i, 0)),
            pl.BlockSpec((_BM, D), lambda i: (i, 0)),
            pl.BlockSpec((D, D), lambda i: (0, 0)),
            pl.BlockSpec((1, D), lambda i: (0, 0)),
        ],
        out_specs=pl.BlockSpec((_BM, D), lambda i: (i, 0)),
        out_shape=jax.ShapeDtypeStruct((N, D), jnp.float32),
    )


_combine_relu = _make_combine(True)
_combine_last = _make_combine(False)


def kernel(x, edge_index, edge_type, W1_rel, W1_root, b1,
           W2_rel, W2_root, b2, W3_rel, W3_root, b3):
    src = edge_index[0]
    dst = edge_index[1]
    et = edge_type

    parts = _sc_count(dst, et)
    inv = _sc_inv(parts)
    scale, rid = _sc_scale(inv, src, et, dst)

    def layer(h, W_rel, W_root, b, relu):
        trans = _dense_trans(h, W_rel).reshape(R * N, D)
        agg = _sc_agg(trans, scale, rid, dst)
        comb = _combine_relu if relu else _combine_last
        return comb(agg, agg, h, W_root, b.reshape(1, D))

    u1 = layer(x, W1_rel, W1_root, b1, True)
    u2 = layer(u1, W2_rel, W2_root, b2, True)
    return layer(u2, W3_rel, W3_root, b3, False)


# speedup vs baseline: 1.1270x; 1.1270x over previous
import functools

import numpy as np

import jax
import jax.numpy as jnp
from jax import lax
from jax.experimental import pallas as pl
from jax.experimental.pallas import tpu as pltpu
from jax.experimental.pallas import tpu_sc as plsc

N = 10000
E = 320000
R = 8
D = 128

NC = 2
NS = 16
NW = NC * NS
EPT = E // NW
KTAB = 81920
BC = 2000
BA = 80
PN = 10240

_mesh = plsc.VectorSubcoreMesh(core_axis_name="c", subcore_axis_name="s")


def _zero_flat(ref, nwords):
    z = jnp.zeros((16,), jnp.float32)

    def body(i, carry):
        ref[pl.ds(i * 16, 16)] = z
        return carry

    lax.fori_loop(0, nwords // 16, body, 0)


KR = 640
ECT = E // NS


@functools.partial(
    pl.kernel,
    out_type=(jax.ShapeDtypeStruct((E,), jnp.float32),
              jax.ShapeDtypeStruct((E,), jnp.int32)),
    mesh=_mesh,
    compiler_params=pltpu.CompilerParams(needs_layout_passes=False),
    scratch_types=[
        pltpu.VMEM((KR, 128), jnp.float32),
        pltpu.VMEM((5, 128), jnp.int32),
        pltpu.VMEM((BC,), jnp.int32),
        pltpu.VMEM((BC,), jnp.int32),
        pltpu.VMEM((BC,), jnp.int32),
        pltpu.VMEM((BC,), jnp.float32),
        pltpu.VMEM((BC,), jnp.int32),
        pltpu.VMEM_SHARED((KR, 128), jnp.float32),
    ],
)
def _sc_prep(src_hbm, et_hbm, dst_hbm, scale_hbm, rid_hbm,
             tbl, idr, sbuf, ebuf, dbuf, scblk, ridblk, cnt_sh):
    c = lax.axis_index("c")
    s = lax.axis_index("s")
    wid = s * NC + c
    z = jnp.zeros((16,), jnp.float32)

    def zb(r, carry):
        for k in range(8):
            tbl[r, pl.ds(k * 16, 16)] = z
        return carry

    lax.fori_loop(0, KR, zb, 0)

    @pl.when(s == 0)
    def _zs():
        pltpu.sync_copy(tbl, cnt_sh)

    for jc in range(5):
        base = jnp.full((16,), jc * 128, jnp.int32) + lax.iota(jnp.int32, 16)
        for k in range(8):
            idr[jc, pl.ds(k * 16, 16)] = base + k * 16
    plsc.subcore_barrier()

    ones = jnp.full((16,), 1.0, jnp.float32)
    cbase = s * ECT
    for blk in range(ECT // BC):
        off = cbase + blk * BC
        pltpu.sync_copy(dst_hbm.at[pl.ds(off, BC)], dbuf)
        pltpu.sync_copy(et_hbm.at[pl.ds(off, BC)], ebuf)

        def cbody(i, carry):
            sl = pl.ds(i * 16, 16)
            key = ebuf[sl] * N + dbuf[sl]
            plsc.addupdate_scatter(tbl, [key >> 7, key & 127], ones)
            return carry

        lax.fori_loop(0, BC // 16, cbody, 0)

    for jc in range(5):
        pltpu.sync_copy(tbl.at[pl.ds(jc * 128, 128)], cnt_sh.at[idr.at[jc]],
                        add=True)
    plsc.subcore_barrier()

    pltpu.sync_copy(cnt_sh, tbl)

    def ib(r, carry):
        for k in range(8):
            sl = pl.ds(k * 16, 16)
            tbl[r, sl] = 1.0 / jnp.maximum(tbl[r, sl], 1.0)
        return carry

    lax.fori_loop(0, KR, ib, 0)

    base = wid * EPT
    for blk in range(EPT // BC):
        off = base + blk * BC
        pltpu.sync_copy(src_hbm.at[pl.ds(off, BC)], sbuf)
        pltpu.sync_copy(et_hbm.at[pl.ds(off, BC)], ebuf)
        pltpu.sync_copy(dst_hbm.at[pl.ds(off, BC)], dbuf)

        def ebody(i, carry):
            sl = pl.ds(i * 16, 16)
            tn = ebuf[sl] * N
            key = tn + dbuf[sl]
            scblk[sl] = plsc.load_gather(tbl, [key >> 7, key & 127])
            ridblk[sl] = tn + sbuf[sl]
            return carry

        lax.fori_loop(0, BC // 16, ebody, 0)
        pltpu.sync_copy(scblk, scale_hbm.at[pl.ds(off, BC)])
        pltpu.sync_copy(ridblk, rid_hbm.at[pl.ds(off, BC)])


BB = 128
NBLK = E // BB // NW
SB = 13
SC_CH = NBLK // SB
SE = SB * BB


@functools.partial(
    pl.kernel,
    out_type=jax.ShapeDtypeStruct((NC, PN, 128), jnp.float32),
    mesh=_mesh,
    compiler_params=pltpu.CompilerParams(needs_layout_passes=False),
    scratch_types=[
        pltpu.VMEM((2, BB, 128), jnp.float32),
        pltpu.VMEM((2, SE), jnp.int32),
        pltpu.VMEM((2, SE), jnp.int32),
        pltpu.VMEM((2, SE), jnp.float32),
        pltpu.VMEM((BB,), jnp.int32),
        pltpu.VMEM_SHARED((PN, 128), jnp.float32),
        pltpu.SemaphoreType.DMA,
        pltpu.SemaphoreType.DMA,
        pltpu.SemaphoreType.DMA,
        pltpu.SemaphoreType.DMA,
    ],
)
def _sc_agg(trans_hbm, scale_hbm, rid_hbm, dst_hbm, out_hbm,
            rows, rid_st, dst_st, scale_st, dstb, acc,
            semi0, semi1, semg0, semg1):
    c = lax.axis_index("c")
    s = lax.axis_index("s")
    wid = s * NC + c
    semi = [semi0, semi1]
    semg = [semg0, semg1]

    z = jnp.zeros((16,), jnp.float32)

    def zb(r, carry):
        for k in range(8):
            rows[0, r, pl.ds(k * 16, 16)] = z
        return carry

    lax.fori_loop(0, BB, zb, 0)
    for j in range(5):
        pltpu.sync_copy(rows.at[0], acc.at[pl.ds(s * 640 + j * 128, 128)])
    plsc.subcore_barrier()

    blk0 = wid * NBLK + jnp.minimum(wid, 4)
    e0 = blk0 * BB

    def stage_idx(ci, cb):
        off = e0 + ci * SE
        return (
            pltpu.async_copy(rid_hbm.at[pl.ds(off, SE)], rid_st.at[cb],
                             semi[cb]),
            pltpu.async_copy(dst_hbm.at[pl.ds(off, SE)], dst_st.at[cb],
                             semi[cb]),
            pltpu.async_copy(scale_hbm.at[pl.ds(off, SE)], scale_st.at[cb],
                             semi[cb]),
        )

    def gather(ci, j, p):
        return pltpu.async_copy(
            trans_hbm.at[rid_st.at[ci % 2, pl.ds(j * BB, BB)]],
            rows.at[p], semg[p])

    def scale_rows(cb, j, p):
        def rbody(r2, carry):
            ra = r2 * 2
            rb = ra + 1
            cbv = jnp.full((16,), cb, jnp.int32)
            cola = jnp.full((16,), j * BB, jnp.int32) + ra
            spa = plsc.load_gather(scale_st, [cbv, cola])
            spb = plsc.load_gather(scale_st, [cbv, cola + 1])
            for k in range(8):
                sl = pl.ds(k * 16, 16)
                rows[p, ra, sl] = rows[p, ra, sl] * spa
                rows[p, rb, sl] = rows[p, rb, sl] * spb
            return carry

        lax.fori_loop(0, BB // 2, rbody, 0)

    def scatter(cb, j, p):
        for k in range(8):
            sl = pl.ds(k * 16, 16)
            dstb[sl] = dst_st[cb, pl.ds(j * BB + k * 16, 16)]
        pltpu.sync_copy(rows.at[p], acc.at[dstb], add=True)

    descs = {0: stage_idx(0, 0), 1: stage_idx(1, 1)}
    for ci in range(SC_CH):
        cb = ci % 2
        for d in descs.pop(ci):
            d.wait()
        g = gather(ci, 0, 0)
        for j in range(SB):
            p = j % 2
            g_next = gather(ci, j + 1, 1 - p) if j + 1 < SB else None
            g.wait()
            scale_rows(cb, j, p)
            scatter(cb, j, p)
            g = g_next
        if ci + 2 < SC_CH:
            descs[ci + 2] = stage_idx(ci + 2, cb)

    @pl.when(wid < 4)
    def _extra():
        off = e0 + NBLK * BB
        pltpu.sync_copy(rid_hbm.at[pl.ds(off, BB)], rid_st.at[0, pl.ds(0, BB)])
        pltpu.sync_copy(dst_hbm.at[pl.ds(off, BB)], dstb)
        pltpu.sync_copy(scale_hbm.at[pl.ds(off, BB)],
                        scale_st.at[0, pl.ds(0, BB)])
        pltpu.async_copy(trans_hbm.at[rid_st.at[0, pl.ds(0, BB)]],
                         rows.at[0], semg0).wait()
        scale_rows(0, 0, 0)
        pltpu.sync_copy(rows.at[0], acc.at[dstb], add=True)

    plsc.subcore_barrier()
    for j in range(5):
        r0 = s * 640 + j * 128
        pltpu.sync_copy(acc.at[pl.ds(r0, 128)], out_hbm.at[c, pl.ds(r0, 128)])


_NB = 10
_BM = N // _NB


def _trans_body(x_ref, w_ref, o_ref):
    o_ref[0] = jnp.dot(x_ref[...], w_ref[0],
                       preferred_element_type=jnp.float32)


_dense_trans = pl.pallas_call(
    _trans_body,
    grid=(R, _NB),
    in_specs=[
        pl.BlockSpec((_BM, D), lambda r, i: (i, 0)),
        pl.BlockSpec((1, D, D), lambda r, i: (r, 0, 0)),
    ],
    out_specs=pl.BlockSpec((1, _BM, D), lambda r, i: (r, i, 0)),
    out_shape=jax.ShapeDtypeStruct((R, N, D), jnp.float32),
)


def _comb_body(relu, a0_ref, a1_ref, x_ref, w_ref, b_ref, o_ref):
    acc = (a0_ref[0] + a1_ref[0]
           + jnp.dot(x_ref[...], w_ref[...],
                     preferred_element_type=jnp.float32)
           + b_ref[...])
    if relu:
        acc = jnp.maximum(acc, 0.0)
    o_ref[...] = acc


def _make_combine(relu):
    return pl.pallas_call(
        functools.partial(_comb_body, relu),
        grid=(_NB,),
        in_specs=[
            pl.BlockSpec((1, _BM, D), lambda i: (0, i, 0)),
            pl.BlockSpec((1, _BM, D), lambda i: (1, i, 0)),
            pl.BlockSpec((_BM, D), lambda i: (i, 0)),
            pl.BlockSpec((D, D), lambda i: (0, 0)),
            pl.BlockSpec((1, D), lambda i: (0, 0)),
        ],
        out_specs=pl.BlockSpec((_BM, D), lambda i: (i, 0)),
        out_shape=jax.ShapeDtypeStruct((N, D), jnp.float32),
    )


_combine_relu = _make_combine(True)
_combine_last = _make_combine(False)


def kernel(x, edge_index, edge_type, W1_rel, W1_root, b1,
           W2_rel, W2_root, b2, W3_rel, W3_root, b3):
    src = edge_index[0]
    dst = edge_index[1]
    et = edge_type

    scale, rid = _sc_prep(src, et, dst)

    def layer(h, W_rel, W_root, b, relu):
        trans = _dense_trans(h, W_rel).reshape(R * N, D)
        agg = _sc_agg(trans, scale, rid, dst)
        comb = _combine_relu if relu else _combine_last
        return comb(agg, agg, h, W_root, b.reshape(1, D))

    u1 = layer(x, W1_rel, W1_root, b1, True)
    u2 = layer(u1, W2_rel, W2_root, b2, True)
    return layer(u2, W3_rel, W3_root, b3, False)


# speedup vs baseline: 1.3165x; 1.1681x over previous
import functools

import numpy as np

import jax
import jax.numpy as jnp
from jax import lax
from jax.experimental import pallas as pl
from jax.experimental.pallas import tpu as pltpu
from jax.experimental.pallas import tpu_sc as plsc

N = 10000
E = 320000
R = 8
D = 128

NC = 2
NS = 16
NW = NC * NS
EPT = E // NW
KTAB = 81920
BC = 2000
BA = 80
PN = 10240

_mesh = plsc.VectorSubcoreMesh(core_axis_name="c", subcore_axis_name="s")


def _zero_flat(ref, nwords):
    z = jnp.zeros((16,), jnp.float32)

    def body(i, carry):
        ref[pl.ds(i * 16, 16)] = z
        return carry

    lax.fori_loop(0, nwords // 16, body, 0)


KR = 640
ECT = E // NS


@functools.partial(
    pl.kernel,
    out_type=(jax.ShapeDtypeStruct((E,), jnp.float32),
              jax.ShapeDtypeStruct((E,), jnp.int32)),
    mesh=_mesh,
    compiler_params=pltpu.CompilerParams(needs_layout_passes=False),
    scratch_types=[
        pltpu.VMEM((KR, 128), jnp.float32),
        pltpu.VMEM((5, 128), jnp.int32),
        pltpu.VMEM((BC,), jnp.int32),
        pltpu.VMEM((BC,), jnp.int32),
        pltpu.VMEM((BC,), jnp.int32),
        pltpu.VMEM((BC,), jnp.float32),
        pltpu.VMEM((BC,), jnp.int32),
        pltpu.VMEM_SHARED((KR, 128), jnp.float32),
    ],
)
def _sc_prep(src_hbm, et_hbm, dst_hbm, scale_hbm, rid_hbm,
             tbl, idr, sbuf, ebuf, dbuf, scblk, ridblk, cnt_sh):
    c = lax.axis_index("c")
    s = lax.axis_index("s")
    wid = s * NC + c
    z = jnp.zeros((16,), jnp.float32)

    def zb(r, carry):
        for k in range(8):
            tbl[r, pl.ds(k * 16, 16)] = z
        return carry

    lax.fori_loop(0, KR, zb, 0)

    @pl.when(s == 0)
    def _zs():
        pltpu.sync_copy(tbl, cnt_sh)

    for jc in range(5):
        base = jnp.full((16,), jc * 128, jnp.int32) + lax.iota(jnp.int32, 16)
        for k in range(8):
            idr[jc, pl.ds(k * 16, 16)] = base + k * 16
    plsc.subcore_barrier()

    ones = jnp.full((16,), 1.0, jnp.float32)
    cbase = s * ECT
    for blk in range(ECT // BC):
        off = cbase + blk * BC
        pltpu.sync_copy(dst_hbm.at[pl.ds(off, BC)], dbuf)
        pltpu.sync_copy(et_hbm.at[pl.ds(off, BC)], ebuf)

        def cbody(i, carry):
            sl = pl.ds(i * 16, 16)
            key = ebuf[sl] * N + dbuf[sl]
            plsc.addupdate_scatter(tbl, [key >> 7, key & 127], ones)
            return carry

        lax.fori_loop(0, BC // 16, cbody, 0)

    for jc in range(5):
        pltpu.sync_copy(tbl.at[pl.ds(jc * 128, 128)], cnt_sh.at[idr.at[jc]],
                        add=True)
    plsc.subcore_barrier()

    pltpu.sync_copy(cnt_sh, tbl)

    def ib(r, carry):
        for k in range(8):
            sl = pl.ds(k * 16, 16)
            tbl[r, sl] = 1.0 / jnp.maximum(tbl[r, sl], 1.0)
        return carry

    lax.fori_loop(0, KR, ib, 0)

    base = wid * EPT
    for blk in range(EPT // BC):
        off = base + blk * BC
        pltpu.sync_copy(src_hbm.at[pl.ds(off, BC)], sbuf)
        pltpu.sync_copy(et_hbm.at[pl.ds(off, BC)], ebuf)
        pltpu.sync_copy(dst_hbm.at[pl.ds(off, BC)], dbuf)

        def ebody(i, carry):
            sl = pl.ds(i * 16, 16)
            tn = ebuf[sl] * N
            key = tn + dbuf[sl]
            scblk[sl] = plsc.load_gather(tbl, [key >> 7, key & 127])
            ridblk[sl] = tn + sbuf[sl]
            return carry

        lax.fori_loop(0, BC // 16, ebody, 0)
        pltpu.sync_copy(scblk, scale_hbm.at[pl.ds(off, BC)])
        pltpu.sync_copy(ridblk, rid_hbm.at[pl.ds(off, BC)])


BB = 128
NBLK = E // BB // NW
SB = 13
SC_CH = NBLK // SB
SE = SB * BB


@functools.partial(
    pl.kernel,
    out_type=jax.ShapeDtypeStruct((NC, PN, 128), jnp.float32),
    mesh=_mesh,
    compiler_params=pltpu.CompilerParams(needs_layout_passes=False),
    scratch_types=[
        pltpu.VMEM((2, BB, 128), jnp.float32),
        pltpu.VMEM((2, SE), jnp.int32),
        pltpu.VMEM((2, SE), jnp.int32),
        pltpu.VMEM((2, SE), jnp.float32),
        pltpu.VMEM((BB,), jnp.int32),
        pltpu.VMEM_SHARED((PN, 128), jnp.float32),
        pltpu.SemaphoreType.DMA,
        pltpu.SemaphoreType.DMA,
        pltpu.SemaphoreType.DMA,
        pltpu.SemaphoreType.DMA,
    ],
)
def _sc_agg(trans_hbm, scale_hbm, rid_hbm, dst_hbm, out_hbm,
            rows, rid_st, dst_st, scale_st, dstb, acc,
            semi0, semi1, semg0, semg1):
    c = lax.axis_index("c")
    s = lax.axis_index("s")
    wid = s * NC + c
    semi = [semi0, semi1]
    semg = [semg0, semg1]

    z = jnp.zeros((16,), jnp.float32)

    def zb(r, carry):
        for k in range(8):
            rows[0, r, pl.ds(k * 16, 16)] = z
        return carry

    lax.fori_loop(0, BB, zb, 0)
    for j in range(5):
        pltpu.sync_copy(rows.at[0], acc.at[pl.ds(s * 640 + j * 128, 128)])
    plsc.subcore_barrier()

    blk0 = wid * NBLK + jnp.minimum(wid, 4)
    e0 = blk0 * BB

    def stage_idx(ci, cb):
        off = e0 + ci * SE
        return (
            pltpu.async_copy(rid_hbm.at[pl.ds(off, SE)], rid_st.at[cb],
                             semi[cb]),
            pltpu.async_copy(dst_hbm.at[pl.ds(off, SE)], dst_st.at[cb],
                             semi[cb]),
            pltpu.async_copy(scale_hbm.at[pl.ds(off, SE)], scale_st.at[cb],
                             semi[cb]),
        )

    def gather(ci, j, p):
        return pltpu.async_copy(
            trans_hbm.at[rid_st.at[ci % 2, pl.ds(j * BB, BB)]],
            rows.at[p], semg[p])

    def scale_rows(cb, j, p):
        def rbody(r2, carry):
            ra = r2 * 2
            rb = ra + 1
            cbv = jnp.full((16,), cb, jnp.int32)
            cola = jnp.full((16,), j * BB, jnp.int32) + ra
            spa = plsc.load_gather(scale_st, [cbv, cola])
            spb = plsc.load_gather(scale_st, [cbv, cola + 1])
            for k in range(8):
                sl = pl.ds(k * 16, 16)
                rows[p, ra, sl] = rows[p, ra, sl] * spa
                rows[p, rb, sl] = rows[p, rb, sl] * spb
            return carry

        lax.fori_loop(0, BB // 2, rbody, 0)

    def scatter(cb, j, p):
        for k in range(8):
            sl = pl.ds(k * 16, 16)
            dstb[sl] = dst_st[cb, pl.ds(j * BB + k * 16, 16)]
        pltpu.sync_copy(rows.at[p], acc.at[dstb], add=True)

    descs = {0: stage_idx(0, 0), 1: stage_idx(1, 1)}
    for ci in range(SC_CH):
        cb = ci % 2
        for d in descs.pop(ci):
            d.wait()
        g = gather(ci, 0, 0)
        for j in range(SB):
            p = j % 2
            g_next = gather(ci, j + 1, 1 - p) if j + 1 < SB else None
            g.wait()
            scale_rows(cb, j, p)
            scatter(cb, j, p)
            g = g_next
        if ci + 2 < SC_CH:
            descs[ci + 2] = stage_idx(ci + 2, cb)

    @pl.when(wid < 4)
    def _extra():
        off = e0 + NBLK * BB
        pltpu.sync_copy(rid_hbm.at[pl.ds(off, BB)], rid_st.at[0, pl.ds(0, BB)])
        pltpu.sync_copy(dst_hbm.at[pl.ds(off, BB)], dstb)
        pltpu.sync_copy(scale_hbm.at[pl.ds(off, BB)],
                        scale_st.at[0, pl.ds(0, BB)])
        pltpu.async_copy(trans_hbm.at[rid_st.at[0, pl.ds(0, BB)]],
                         rows.at[0], semg0).wait()
        scale_rows(0, 0, 0)
        pltpu.sync_copy(rows.at[0], acc.at[dstb], add=True)

    plsc.subcore_barrier()
    for j in range(5):
        r0 = s * 640 + j * 128
        pltpu.sync_copy(acc.at[pl.ds(r0, 128)], out_hbm.at[c, pl.ds(r0, 128)])


_NB = 10
_BM = N // _NB


def _trans_body(x_ref, w_ref, o_ref):
    o_ref[0] = jnp.dot(x_ref[...], w_ref[0],
                       preferred_element_type=jnp.float32)


_dense_trans = pl.pallas_call(
    _trans_body,
    grid=(R, _NB),
    in_specs=[
        pl.BlockSpec((_BM, D), lambda r, i: (i, 0)),
        pl.BlockSpec((1, D, D), lambda r, i: (r, 0, 0)),
    ],
    out_specs=pl.BlockSpec((1, _BM, D), lambda r, i: (r, i, 0)),
    out_shape=jax.ShapeDtypeStruct((R, N, D), jnp.float32),
)


def _comb_body(relu, a0_ref, a1_ref, x_ref, w_ref, b_ref, o_ref):
    acc = (a0_ref[0] + a1_ref[0]
           + jnp.dot(x_ref[...], w_ref[...],
                     preferred_element_type=jnp.float32)
           + b_ref[...])
    if relu:
        acc = jnp.maximum(acc, 0.0)
    o_ref[...] = acc


def _make_combine(relu):
    return pl.pallas_call(
        functools.partial(_comb_body, relu),
        grid=(_NB,),
        in_specs=[
            pl.BlockSpec((1, _BM, D), lambda i: (0, i, 0)),
            pl.BlockSpec((1, _BM, D), lambda i: (1, i, 0)),
            pl.BlockSpec((_BM, D), lambda i: (i, 0)),
            pl.BlockSpec((D, D), lambda i: (0, 0)),
            pl.BlockSpec((1, D), lambda i: (0, 0)),
        ],
        out_specs=pl.BlockSpec((_BM, D), lambda i: (i, 0)),
        out_shape=jax.ShapeDtypeStruct((N, D), jnp.float32),
    )


_combine_last = _make_combine(False)


def _fused_body(a0_ref, a1_ref, x_ref, wr_ref, b_ref, wn_ref, oh_ref, ot_ref):
    h = (a0_ref[0] + a1_ref[0]
         + jnp.dot(x_ref[...], wr_ref[...],
                   preferred_element_type=jnp.float32)
         + b_ref[...])
    h = jnp.maximum(h, 0.0)
    oh_ref[...] = h
    for r in range(R):
        ot_ref[r] = jnp.dot(h, wn_ref[r], preferred_element_type=jnp.float32)


_dense_fused = pl.pallas_call(
    _fused_body,
    grid=(_NB,),
    in_specs=[
        pl.BlockSpec((1, _BM, D), lambda i: (0, i, 0)),
        pl.BlockSpec((1, _BM, D), lambda i: (1, i, 0)),
        pl.BlockSpec((_BM, D), lambda i: (i, 0)),
        pl.BlockSpec((D, D), lambda i: (0, 0)),
        pl.BlockSpec((1, D), lambda i: (0, 0)),
        pl.BlockSpec((R, D, D), lambda i: (0, 0, 0)),
    ],
    out_specs=[
        pl.BlockSpec((_BM, D), lambda i: (i, 0)),
        pl.BlockSpec((R, _BM, D), lambda i: (0, i, 0)),
    ],
    out_shape=[jax.ShapeDtypeStruct((N, D), jnp.float32),
               jax.ShapeDtypeStruct((R, N, D), jnp.float32)],
)


def kernel(x, edge_index, edge_type, W1_rel, W1_root, b1,
           W2_rel, W2_root, b2, W3_rel, W3_root, b3):
    src = edge_index[0]
    dst = edge_index[1]
    et = edge_type

    scale, rid = _sc_prep(src, et, dst)

    t1 = _dense_trans(x, W1_rel).reshape(R * N, D)
    a1 = _sc_agg(t1, scale, rid, dst)
    h1, t2 = _dense_fused(a1, a1, x, W1_root, b1.reshape(1, D), W2_rel)
    a2 = _sc_agg(t2.reshape(R * N, D), scale, rid, dst)
    h2, t3 = _dense_fused(a2, a2, h1, W2_root, b2.reshape(1, D), W3_rel)
    a3 = _sc_agg(t3.reshape(R * N, D), scale, rid, dst)
    return _combine_last(a3, a3, h2, W3_root, b3.reshape(1, D))
